# baseline (speedup 1.0000x reference)
import jax
import jax.numpy as jnp
from jax import lax
from jax.experimental import pallas as pl
from jax.experimental.pallas import tpu as pltpu
from jax.experimental.pallas import tpu_sc as plsc

_B, _S, _D = 64, 4096, 128
_NC, _NS = 2, 16
_NW = _NC * _NS
_ROWS = _B * _S
_CH = 128
_CPB = _S // _CH
_NK = _B
_NJ = _D // 16
_G = _CH // 16
_MAXP = 500.0
_MINP = 0.0


def _splat(vec16, lane):
    return lax.gather(
        vec16, jnp.full((16, 1), lane, jnp.int32),
        lax.GatherDimensionNumbers(
            offset_dims=(), collapsed_slice_dims=(0,), start_index_map=(0,)),
        slice_sizes=(1,),
        mode=lax.GatherScatterMode.PROMISE_IN_BOUNDS)


def _sc_body(x_hbm, w_hbm, sol_hbm, par_hbm, out_hbm,
             xb0, xb1, yb0, yb1, wb0, wb1, xas, solbuf, parbuf,
             sin0, sin1, sout0, sout1):
    cid = lax.axis_index("c")
    sid = lax.axis_index("s")
    wid = sid * _NC + cid
    pltpu.sync_copy(par_hbm, parbuf)
    pltpu.sync_copy(sol_hbm, solbuf)
    pvec = parbuf[...]
    coef = pvec[0]
    scale = pvec[1]
    iota128 = jnp.arange(16, dtype=jnp.int32) * _D

    xbufs, ybufs, wbufs = (xb0, xb1), (yb0, yb1), (wb0, wb1)
    sins, souts = (sin0, sin1), (sout0, sout1)

    def chunk_base(k):
        return pl.multiple_of((k * _CPB + wid) * (_CH * _D), _CH * _D)

    def issue_in(k, b):
        base = chunk_base(k)
        pltpu.async_copy(x_hbm.at[pl.ds(base, _CH * _D)], xbufs[b], sins[b])
        pltpu.async_copy(w_hbm.at[pl.ds(pl.multiple_of(base // _D, _CH), _CH)],
                         wbufs[b], sins[b])

    issue_in(0, 0)

    def outer(ko, carry):
        for bsel in (0, 1):
            k = ko * 2 + bsel
            xb, yb, wb = xbufs[bsel], ybufs[bsel], wbufs[bsel]
            pltpu.make_async_copy(x_hbm.at[pl.ds(0, _CH * _D)], xb, sins[bsel]).wait()
            pltpu.make_async_copy(w_hbm.at[pl.ds(0, _CH)], wb, sins[bsel]).wait()

            @pl.when(k + 1 < _NK)
            def _():
                issue_in(k + 1, bsel ^ 1)

            s16 = solbuf[pl.ds(pl.multiple_of((k // 16) * 16, 16), 16)]
            is_sol = jnp.max(_splat(s16, k % 16)) > 0.5

            @pl.when(k >= 2)
            def _():
                pltpu.make_async_copy(yb, out_hbm.at[pl.ds(0, _CH * _D)],
                                      souts[bsel]).wait()

            @pl.when(is_sol)
            def _():
                def group(g, c):
                    wv = wb[pl.ds(pl.multiple_of(g * 16, 16), 16)]
                    af = coef * jnp.minimum(jnp.maximum(wv * scale, 0.01), 1.0)
                    idx0 = g * (16 * _D) + iota128
                    acc = [None] * 4
                    mx = [None] * 4
                    mn = [None] * 4
                    for j in range(_D):
                        v = plsc.load_gather(xb, [idx0 + j]) * af
                        xas[j, :] = v
                        q = j & 3
                        if acc[q] is None:
                            acc[q] = v
                            mx[q] = v
                            mn[q] = v
                        else:
                            acc[q] = acc[q] + v
                            mx[q] = jnp.maximum(mx[q], v)
                            mn[q] = jnp.minimum(mn[q], v)
                    target = (acc[0] + acc[1]) + (acc[2] + acc[3])
                    mxv = jnp.maximum(jnp.maximum(mx[0], mx[1]),
                                      jnp.maximum(mx[2], mx[3]))
                    mnv = jnp.minimum(jnp.minimum(mn[0], mn[1]),
                                      jnp.minimum(mn[2], mn[3]))
                    rng = jnp.maximum(mxv - mnv, 1.0)
                    lmin = -rng
                    lmax = rng
                    for _i in range(5):
                        lmid = (lmin + lmax) * 0.5
                        sacc = [None] * 4
                        for j in range(_D):
                            y = jnp.minimum(
                                jnp.maximum(xas[j, :] - lmid, _MINP), _MAXP)
                            q = j & 3
                            sacc[q] = y if sacc[q] is None else sacc[q] + y
                        total = (sacc[0] + sacc[1]) + (sacc[2] + sacc[3])
                        diff = total - target
                        conv = jnp.abs(diff) < 0.1
                        lmin = jnp.where((total > target) & (~conv), lmid, lmin)
                        lmax = jnp.where((total <= target) & (~conv), lmid, lmax)
                    lam = (lmin + lmax) * 0.5
                    for j in range(_D):
                        y = jnp.minimum(jnp.maximum(xas[j, :] - lam, _MINP), _MAXP)
                        plsc.store_scatter(yb, [idx0 + j], y)
                    return c

                lax.fori_loop(0, _G, group, 0)

            @pl.when(jnp.logical_not(is_sol))
            def _():
                @plsc.parallel_loop(0, _CH, 1, unroll=4)
                def row_relu(r):
                    ro = pl.multiple_of(r * _D, _D)
                    for j in range(_NJ):
                        yb[pl.ds(ro + j * 16, 16)] = jnp.maximum(
                            xb[pl.ds(ro + j * 16, 16)], 0.0)

            pltpu.async_copy(yb, out_hbm.at[pl.ds(chunk_base(k), _CH * _D)],
                             souts[bsel])
        return carry

    lax.fori_loop(0, _NK // 2, outer, 0)
    pltpu.make_async_copy(yb0, out_hbm.at[pl.ds(0, _CH * _D)], souts[0]).wait()
    pltpu.make_async_copy(yb1, out_hbm.at[pl.ds(0, _CH * _D)], souts[1]).wait()


def kernel(x, weather_data, is_solar, unit_ids, c_prime, alpha, alpha_prime, ssrd_scale, A, eta):
    del unit_ids
    coef = c_prime * A * eta / (alpha + alpha_prime) * _MAXP
    params = jnp.zeros((16,), jnp.float32)
    params = params.at[0].set(coef).at[1].set(ssrd_scale.astype(jnp.float32))
    solf = (is_solar[:, 0] == 1).astype(jnp.float32)
    xf = x.reshape(_ROWS * _D)
    wf = weather_data.reshape(_ROWS)
    mesh = plsc.VectorSubcoreMesh(core_axis_name="c", subcore_axis_name="s")
    out = pl.kernel(
        _sc_body,
        out_type=jax.ShapeDtypeStruct((_ROWS * _D,), jnp.float32),
        mesh=mesh,
        compiler_params=pltpu.CompilerParams(needs_layout_passes=False),
        scratch_types=[
            pltpu.VMEM((_CH * _D,), jnp.float32),
            pltpu.VMEM((_CH * _D,), jnp.float32),
            pltpu.VMEM((_CH * _D,), jnp.float32),
            pltpu.VMEM((_CH * _D,), jnp.float32),
            pltpu.VMEM((_CH,), jnp.float32),
            pltpu.VMEM((_CH,), jnp.float32),
            pltpu.VMEM((_D, 16), jnp.float32),
            pltpu.VMEM((_B,), jnp.float32),
            pltpu.VMEM((16,), jnp.float32),
            pltpu.SemaphoreType.DMA,
            pltpu.SemaphoreType.DMA,
            pltpu.SemaphoreType.DMA,
            pltpu.SemaphoreType.DMA,
        ],
    )(xf, wf, solf, params)
    return out.reshape(_B, _S, _D)


# speedup vs baseline: 1.0667x; 1.0667x over previous
import jax
import jax.numpy as jnp
from jax import lax
from jax.experimental import pallas as pl
from jax.experimental.pallas import tpu as pltpu
from jax.experimental.pallas import tpu_sc as plsc

_B, _S, _D = 64, 4096, 128
_NC, _NS = 2, 16
_NW = _NC * _NS
_ROWS = _B * _S
_CH = 128
_CPB = _S // _CH
_NK = _B
_NJ = _D // 16
_G = _CH // 16
_MAXP = 500.0
_MINP = 0.0


def _splat(vec16, lane):
    return lax.gather(
        vec16, jnp.full((16, 1), lane, jnp.int32),
        lax.GatherDimensionNumbers(
            offset_dims=(), collapsed_slice_dims=(0,), start_index_map=(0,)),
        slice_sizes=(1,),
        mode=lax.GatherScatterMode.PROMISE_IN_BOUNDS)


def _sc_body(x_hbm, w_hbm, sol_hbm, par_hbm, out_hbm,
             xb0, xb1, yb0, yb1, wb0, wb1, xas, solbuf, parbuf,
             sin0, sin1, sout0, sout1):
    cid = lax.axis_index("c")
    sid = lax.axis_index("s")
    wid = sid * _NC + cid
    pltpu.sync_copy(par_hbm, parbuf)
    pltpu.sync_copy(sol_hbm, solbuf)
    pvec = parbuf[...]
    coef = pvec[0]
    scale = pvec[1]
    iota16 = jnp.arange(16, dtype=jnp.int32)
    cb16 = iota16 * 16
    cb128 = iota16 * _D

    xbufs, ybufs, wbufs = (xb0, xb1), (yb0, yb1), (wb0, wb1)
    sins, souts = (sin0, sin1), (sout0, sout1)

    def chunk_base(k):
        return pl.multiple_of((k * _CPB + wid) * (_CH * _D), _CH * _D)

    def issue_in(k, b):
        base = chunk_base(k)
        pltpu.async_copy(x_hbm.at[pl.ds(base, _CH * _D)], xbufs[b], sins[b])
        pltpu.async_copy(w_hbm.at[pl.ds(pl.multiple_of(base // _D, _CH), _CH)],
                         wbufs[b], sins[b])

    issue_in(0, 0)

    def outer(ko, carry):
        for bsel in (0, 1):
            k = ko * 2 + bsel
            xb, yb, wb = xbufs[bsel], ybufs[bsel], wbufs[bsel]
            pltpu.make_async_copy(x_hbm.at[pl.ds(0, _CH * _D)], xb, sins[bsel]).wait()
            pltpu.make_async_copy(w_hbm.at[pl.ds(0, _CH)], wb, sins[bsel]).wait()

            @pl.when(k + 1 < _NK)
            def _():
                issue_in(k + 1, bsel ^ 1)

            s16 = solbuf[pl.ds(pl.multiple_of((k // 16) * 16, 16), 16)]
            is_sol = jnp.max(_splat(s16, k % 16)) > 0.5

            @pl.when(k >= 2)
            def _():
                pltpu.make_async_copy(yb, out_hbm.at[pl.ds(0, _CH * _D)],
                                      souts[bsel]).wait()

            @pl.when(is_sol)
            def _():
                def group(g, c):
                    wv = wb[pl.ds(pl.multiple_of(g * 16, 16), 16)]
                    af16 = coef * jnp.minimum(jnp.maximum(wv * scale, 0.01), 1.0)
                    go = pl.multiple_of(g * (16 * _D), 16 * _D)

                    for r in range(16):
                        af_r = af16[r]
                        ro = go + r * _D
                        for j in range(_NJ):
                            v = xb[pl.ds(ro + j * 16, 16)] * af_r
                            plsc.store_scatter(xas, [cb16 + (j * 256 + r)], v)

                    acc = [None] * 8
                    mx = [None] * 8
                    mn = [None] * 8
                    for jj in range(_D):
                        v = xas[pl.ds(jj * 16, 16)]
                        q = jj & 7
                        if acc[q] is None:
                            acc[q], mx[q], mn[q] = v, v, v
                        else:
                            acc[q] = acc[q] + v
                            mx[q] = jnp.maximum(mx[q], v)
                            mn[q] = jnp.minimum(mn[q], v)
                    for st in (acc, mx, mn):
                        for w in (4, 2, 1):
                            for i in range(w):
                                if st is acc:
                                    st[i] = st[i] + st[i + w]
                                elif st is mx:
                                    st[i] = jnp.maximum(st[i], st[i + w])
                                else:
                                    st[i] = jnp.minimum(st[i], st[i + w])
                    target = acc[0]
                    rng = jnp.maximum(mx[0] - mn[0], 1.0)
                    lmin = -rng
                    lmax = rng

                    for _i in range(5):
                        lmid = (lmin + lmax) * 0.5
                        sacc = [None] * 8
                        for jj in range(_D):
                            v = xas[pl.ds(jj * 16, 16)]
                            y = jnp.minimum(jnp.maximum(v - lmid, _MINP), _MAXP)
                            q = jj & 7
                            sacc[q] = y if sacc[q] is None else sacc[q] + y
                        for w in (4, 2, 1):
                            for i in range(w):
                                sacc[i] = sacc[i] + sacc[i + w]
                        total = sacc[0]
                        diff = total - target
                        conv = jnp.abs(diff) < 0.1
                        lmin = jnp.where((total > target) & (~conv), lmid, lmin)
                        lmax = jnp.where((total <= target) & (~conv), lmid, lmax)
                    lam = (lmin + lmax) * 0.5

                    gbase = g * (16 * _D)
                    for jj in range(_D):
                        v = xas[pl.ds(jj * 16, 16)]
                        y = jnp.minimum(jnp.maximum(v - lam, _MINP), _MAXP)
                        plsc.store_scatter(yb, [cb128 + (gbase + jj)], y)
                    return c

                lax.fori_loop(0, _G, group, 0)

            @pl.when(jnp.logical_not(is_sol))
            def _():
                @plsc.parallel_loop(0, _CH, 1, unroll=4)
                def row_relu(r):
                    ro = pl.multiple_of(r * _D, _D)
                    for j in range(_NJ):
                        yb[pl.ds(ro + j * 16, 16)] = jnp.maximum(
                            xb[pl.ds(ro + j * 16, 16)], 0.0)

            pltpu.async_copy(yb, out_hbm.at[pl.ds(chunk_base(k), _CH * _D)],
                             souts[bsel])
        return carry

    lax.fori_loop(0, _NK // 2, outer, 0)
    pltpu.make_async_copy(yb0, out_hbm.at[pl.ds(0, _CH * _D)], souts[0]).wait()
    pltpu.make_async_copy(yb1, out_hbm.at[pl.ds(0, _CH * _D)], souts[1]).wait()


def kernel(x, weather_data, is_solar, unit_ids, c_prime, alpha, alpha_prime, ssrd_scale, A, eta):
    del unit_ids
    coef = c_prime * A * eta / (alpha + alpha_prime) * _MAXP
    params = jnp.zeros((16,), jnp.float32)
    params = params.at[0].set(coef).at[1].set(ssrd_scale.astype(jnp.float32))
    solf = (is_solar[:, 0] == 1).astype(jnp.float32)
    xf = x.reshape(_ROWS * _D)
    wf = weather_data.reshape(_ROWS)
    mesh = plsc.VectorSubcoreMesh(core_axis_name="c", subcore_axis_name="s")
    out = pl.kernel(
        _sc_body,
        out_type=jax.ShapeDtypeStruct((_ROWS * _D,), jnp.float32),
        mesh=mesh,
        compiler_params=pltpu.CompilerParams(needs_layout_passes=False),
        scratch_types=[
            pltpu.VMEM((_CH * _D,), jnp.float32),
            pltpu.VMEM((_CH * _D,), jnp.float32),
            pltpu.VMEM((_CH * _D,), jnp.float32),
            pltpu.VMEM((_CH * _D,), jnp.float32),
            pltpu.VMEM((_CH,), jnp.float32),
            pltpu.VMEM((_CH,), jnp.float32),
            pltpu.VMEM((16 * _D,), jnp.float32),
            pltpu.VMEM((_B,), jnp.float32),
            pltpu.VMEM((16,), jnp.float32),
            pltpu.SemaphoreType.DMA,
            pltpu.SemaphoreType.DMA,
            pltpu.SemaphoreType.DMA,
            pltpu.SemaphoreType.DMA,
        ],
    )(xf, wf, solf, params)
    return out.reshape(_B, _S, _D)


# speedup vs baseline: 1.1303x; 1.0596x over previous
import jax
import jax.numpy as jnp
from jax import lax
from jax.experimental import pallas as pl
from jax.experimental.pallas import tpu as pltpu
from jax.experimental.pallas import tpu_sc as plsc

_B, _S, _D = 64, 4096, 128
_NC, _NS = 2, 16
_NW = _NC * _NS
_ROWS = _B * _S
_CH = 128
_CPB = _S // _CH
_NK = _B
_NJ = _D // 16
_G = _CH // 16
_MAXP = 500.0
_MINP = 0.0


def _splat(vec16, lane):
    return lax.gather(
        vec16, jnp.full((16, 1), lane, jnp.int32),
        lax.GatherDimensionNumbers(
            offset_dims=(), collapsed_slice_dims=(0,), start_index_map=(0,)),
        slice_sizes=(1,),
        mode=lax.GatherScatterMode.PROMISE_IN_BOUNDS)


def _sc_body(x_hbm, w_hbm, sol_hbm, par_hbm, out_hbm,
             xb0, xb1, yb0, yb1, wb0, wb1, xas, solbuf, parbuf,
             sin0, sin1, sout0, sout1):
    cid = lax.axis_index("c")
    sid = lax.axis_index("s")
    wid = sid * _NC + cid
    pltpu.sync_copy(par_hbm, parbuf)
    pltpu.sync_copy(sol_hbm, solbuf)
    pvec = parbuf[...]
    coef = pvec[0]
    scale = pvec[1]
    iota16 = jnp.arange(16, dtype=jnp.int32)
    cb16 = iota16 * 16
    cb128 = iota16 * _D

    xbufs, ybufs, wbufs = (xb0, xb1), (yb0, yb1), (wb0, wb1)
    sins, souts = (sin0, sin1), (sout0, sout1)

    def chunk_base(k):
        return pl.multiple_of((k * _CPB + wid) * (_CH * _D), _CH * _D)

    def issue_in(k, b):
        base = chunk_base(k)
        pltpu.async_copy(x_hbm.at[pl.ds(base, _CH * _D)], xbufs[b], sins[b])
        pltpu.async_copy(w_hbm.at[pl.ds(pl.multiple_of(base // _D, _CH), _CH)],
                         wbufs[b], sins[b])

    issue_in(0, 0)

    def outer(ko, carry):
        for bsel in (0, 1):
            k = ko * 2 + bsel
            xb, yb, wb = xbufs[bsel], ybufs[bsel], wbufs[bsel]
            pltpu.make_async_copy(x_hbm.at[pl.ds(0, _CH * _D)], xb, sins[bsel]).wait()
            pltpu.make_async_copy(w_hbm.at[pl.ds(0, _CH)], wb, sins[bsel]).wait()

            @pl.when(k + 1 < _NK)
            def _():
                issue_in(k + 1, bsel ^ 1)

            s16 = solbuf[pl.ds(pl.multiple_of((k // 16) * 16, 16), 16)]
            is_sol = jnp.max(_splat(s16, k % 16)) > 0.5

            @pl.when(k >= 2)
            def _():
                pltpu.make_async_copy(yb, out_hbm.at[pl.ds(0, _CH * _D)],
                                      souts[bsel]).wait()

            @pl.when(is_sol)
            def _():
                def group(g, c):
                    wv = wb[pl.ds(pl.multiple_of(g * 16, 16), 16)]
                    af16 = coef * jnp.minimum(jnp.maximum(wv * scale, 0.01), 1.0)
                    go = pl.multiple_of(g * (16 * _D), 16 * _D)

                    acc = [None] * 8
                    mx = [None] * 8
                    mn = [None] * 8
                    for t in range(16):
                        bidx = (cb128 + ((iota16 + t) & 15)) + go
                        for j in range(_NJ):
                            v = plsc.load_gather(xb, [bidx + j * 16]) * af16
                            xas[pl.ds((t * _NJ + j) * 16, 16)] = v
                            if acc[j] is None:
                                acc[j], mx[j], mn[j] = v, v, v
                            else:
                                acc[j] = acc[j] + v
                                mx[j] = jnp.maximum(mx[j], v)
                                mn[j] = jnp.minimum(mn[j], v)
                    for w in (4, 2, 1):
                        for i in range(w):
                            acc[i] = acc[i] + acc[i + w]
                            mx[i] = jnp.maximum(mx[i], mx[i + w])
                            mn[i] = jnp.minimum(mn[i], mn[i + w])
                    target = acc[0]
                    rng = jnp.maximum(mx[0] - mn[0], 1.0)
                    lmin = -rng
                    lmax = rng

                    for _i in range(5):
                        lmid = (lmin + lmax) * 0.5
                        sacc = [None] * 8
                        for jj in range(_D):
                            v = xas[pl.ds(jj * 16, 16)]
                            y = jnp.minimum(jnp.maximum(v - lmid, _MINP), _MAXP)
                            q = jj & 7
                            sacc[q] = y if sacc[q] is None else sacc[q] + y
                        for w in (4, 2, 1):
                            for i in range(w):
                                sacc[i] = sacc[i] + sacc[i + w]
                        total = sacc[0]
                        diff = total - target
                        conv = jnp.abs(diff) < 0.1
                        lmin = jnp.where((total > target) & (~conv), lmid, lmin)
                        lmax = jnp.where((total <= target) & (~conv), lmid, lmax)
                    lam = (lmin + lmax) * 0.5

                    for t in range(16):
                        bidx = (cb128 + ((iota16 + t) & 15)) + go
                        for j in range(_NJ):
                            v = xas[pl.ds((t * _NJ + j) * 16, 16)]
                            y = jnp.minimum(jnp.maximum(v - lam, _MINP), _MAXP)
                            plsc.store_scatter(yb, [bidx + j * 16], y)
                    return c

                lax.fori_loop(0, _G, group, 0)

            @pl.when(jnp.logical_not(is_sol))
            def _():
                @plsc.parallel_loop(0, _CH, 1, unroll=4)
                def row_relu(r):
                    ro = pl.multiple_of(r * _D, _D)
                    for j in range(_NJ):
                        yb[pl.ds(ro + j * 16, 16)] = jnp.maximum(
                            xb[pl.ds(ro + j * 16, 16)], 0.0)

            pltpu.async_copy(yb, out_hbm.at[pl.ds(chunk_base(k), _CH * _D)],
                             souts[bsel])
        return carry

    lax.fori_loop(0, _NK // 2, outer, 0)
    pltpu.make_async_copy(yb0, out_hbm.at[pl.ds(0, _CH * _D)], souts[0]).wait()
    pltpu.make_async_copy(yb1, out_hbm.at[pl.ds(0, _CH * _D)], souts[1]).wait()


def kernel(x, weather_data, is_solar, unit_ids, c_prime, alpha, alpha_prime, ssrd_scale, A, eta):
    del unit_ids
    coef = c_prime * A * eta / (alpha + alpha_prime) * _MAXP
    params = jnp.zeros((16,), jnp.float32)
    params = params.at[0].set(coef).at[1].set(ssrd_scale.astype(jnp.float32))
    solf = (is_solar[:, 0] == 1).astype(jnp.float32)
    xf = x.reshape(_ROWS * _D)
    wf = weather_data.reshape(_ROWS)
    mesh = plsc.VectorSubcoreMesh(core_axis_name="c", subcore_axis_name="s")
    out = pl.kernel(
        _sc_body,
        out_type=jax.ShapeDtypeStruct((_ROWS * _D,), jnp.float32),
        mesh=mesh,
        compiler_params=pltpu.CompilerParams(needs_layout_passes=False),
        scratch_types=[
            pltpu.VMEM((_CH * _D,), jnp.float32),
            pltpu.VMEM((_CH * _D,), jnp.float32),
            pltpu.VMEM((_CH * _D,), jnp.float32),
            pltpu.VMEM((_CH * _D,), jnp.float32),
            pltpu.VMEM((_CH,), jnp.float32),
            pltpu.VMEM((_CH,), jnp.float32),
            pltpu.VMEM((16 * _D,), jnp.float32),
            pltpu.VMEM((_B,), jnp.float32),
            pltpu.VMEM((16,), jnp.float32),
            pltpu.SemaphoreType.DMA,
            pltpu.SemaphoreType.DMA,
            pltpu.SemaphoreType.DMA,
            pltpu.SemaphoreType.DMA,
        ],
    )(xf, wf, solf, params)
    return out.reshape(_B, _S, _D)
